# baseline (speedup 1.0000x reference)
import functools

import jax
import jax.numpy as jnp
from jax import lax
from jax.experimental import pallas as pl
from jax.experimental.pallas import tpu as pltpu
from jax.experimental.pallas import tpu_sc as plsc

_B = 16384
_ZD = 128
_NC, _NS = 2, 16
_NW = _NC * _NS
_BPW = _B // _NW
_CH = 128
_NCH = _BPW // _CH


def _sc_gather(ztab, iz):
    mesh = plsc.VectorSubcoreMesh(core_axis_name="c", subcore_axis_name="s")

    @functools.partial(
        pl.kernel, mesh=mesh,
        out_type=jax.ShapeDtypeStruct((_B, _ZD), jnp.float32),
        scratch_types=[
            pltpu.VMEM((_NCH, _CH), jnp.int32),
            pltpu.VMEM((_BPW, _ZD), jnp.float32),
            pltpu.SemaphoreType.DMA,
        ])
    def k(ztab_h, iz_h, zo_h, ziv, zrows, sem):
        wid = lax.axis_index("s") * _NC + lax.axis_index("c")
        base = wid * _BPW
        pltpu.sync_copy(iz_h.at[pl.ds(wid * _NCH, _NCH)], ziv)
        handles = [
            pltpu.async_copy(ztab_h.at[ziv.at[j]],
                             zrows.at[pl.ds(j * _CH, _CH)], sem)
            for j in range(_NCH)
        ]
        for h in handles:
            h.wait()
        pltpu.sync_copy(zrows, zo_h.at[pl.ds(base, _BPW)])

    return k(ztab, iz)


_TB = 2048


def _mlp_body(z_ref, ip_ref, it_ref, w1z, ptab, w1p, ttab, w1t, b1r, w2,
              b2r, w3, b3r, o_ref):
    dot = functools.partial(jnp.dot, preferred_element_type=jnp.float32,
                            precision=lax.Precision.HIGHEST)
    iota16 = lax.broadcasted_iota(jnp.int32, (1, 16), 1)
    one_p = (ip_ref[...] == iota16).astype(jnp.float32)
    one_t = (it_ref[...] == iota16).astype(jnp.float32)
    acc = dot(z_ref[...], w1z[...])
    acc += dot(dot(one_p, ptab[...]), w1p[...])
    acc += dot(dot(one_t, ttab[...]), w1t[...])
    h1 = jnp.maximum(acc + b1r[...], 0.0)
    h2 = jnp.maximum(dot(h1, w2[...]) + b2r[...], 0.0)
    o_ref[...] = dot(h2, w3[...]) + b3r[...]


def _tc_mlp(z, ip, it, w1z, ptab, w1p, ttab, w1t, b1, w2, b2, w3, b3):
    grid = (_B // _TB,)
    batch_spec = lambda cols: pl.BlockSpec((_TB, cols), lambda i: (i, 0))
    full = lambda a: pl.BlockSpec(a.shape, lambda i: (0, 0))
    return pl.pallas_call(
        _mlp_body,
        grid=grid,
        in_specs=[batch_spec(_ZD), batch_spec(1), batch_spec(1),
                  full(w1z), full(ptab), full(w1p), full(ttab), full(w1t),
                  full(b1), full(w2), full(b2), full(w3), full(b3)],
        out_specs=batch_spec(64),
        out_shape=jax.ShapeDtypeStruct((_B, 64), jnp.float32),
    )(z, ip, it, w1z, ptab, w1p, ttab, w1t, b1, w2, b2, w3, b3)


def kernel(zip_code_id, project_type_id, trade_needed_id, zip_emb, pt_emb,
           tr_emb, W1, b1, W2, b2, W3, b3):
    ztab = jnp.pad(zip_emb, ((0, 0), (0, _ZD - 24)))
    iz = zip_code_id.astype(jnp.int32).reshape(_B // _CH, _CH)
    z = _sc_gather(ztab, iz)
    ip = project_type_id.astype(jnp.int32).reshape(_B, 1)
    it = trade_needed_id.astype(jnp.int32).reshape(_B, 1)
    w1z = jnp.pad(W1[0:24], ((0, _ZD - 24), (0, 0)))
    ptab = jnp.pad(pt_emb, ((0, 16 - 9), (0, 16 - 12)))
    w1p = jnp.pad(W1[24:36], ((0, 4), (0, 0)))
    ttab = jnp.pad(tr_emb, ((0, 16 - 11), (0, 0)))
    w1t = W1[36:52]
    return _tc_mlp(z, ip, it, w1z, ptab, w1p, ttab, w1t,
                   b1.reshape(1, 512), W2, b2.reshape(1, 128), W3,
                   b3.reshape(1, 64))


# speedup vs baseline: 2.5251x; 2.5251x over previous
import functools

import jax
import jax.numpy as jnp
from jax import lax
from jax.experimental import pallas as pl
from jax.experimental.pallas import tpu as pltpu
from jax.experimental.pallas import tpu_sc as plsc

_B = 16384
_ZD = 128
_NC, _NS = 2, 16
_NW = _NC * _NS
_BPW = _B // _NW
_CH = 128
_NCH = _BPW // _CH


def _sc_gather(ztab, iz):
    mesh = plsc.VectorSubcoreMesh(core_axis_name="c", subcore_axis_name="s")

    @functools.partial(
        pl.kernel, mesh=mesh,
        out_type=jax.ShapeDtypeStruct((_B, _ZD), jnp.float32),
        scratch_types=[
            pltpu.VMEM((_NCH, _CH), jnp.int32),
            pltpu.VMEM((_BPW, _ZD), jnp.float32),
            pltpu.SemaphoreType.DMA,
        ])
    def k(ztab_h, iz_h, zo_h, ziv, zrows, sem):
        wid = lax.axis_index("s") * _NC + lax.axis_index("c")
        base = wid * _BPW
        pltpu.sync_copy(iz_h.at[pl.ds(wid * _NCH, _NCH)], ziv)
        handles = [
            pltpu.async_copy(ztab_h.at[ziv.at[j]],
                             zrows.at[pl.ds(j * _CH, _CH)], sem)
            for j in range(_NCH)
        ]
        for h in handles:
            h.wait()
        pltpu.sync_copy(zrows, zo_h.at[pl.ds(base, _BPW)])

    return k(ztab, iz)


_TB = 2048


def _mlp_body(z_ref, ip_ref, it_ref, w1z, ptab, w1p, ttab, w1t, b1r, w2,
              b2r, w3, b3r, o_ref):
    dot = functools.partial(jnp.dot, preferred_element_type=jnp.float32,
                            precision=lax.Precision.DEFAULT)
    hdot = functools.partial(jnp.dot, preferred_element_type=jnp.float32,
                             precision=lax.Precision.HIGHEST)
    iota16 = lax.broadcasted_iota(jnp.int32, (1, 16), 1)
    one_p = (ip_ref[...] == iota16).astype(jnp.float32)
    one_t = (it_ref[...] == iota16).astype(jnp.float32)
    acc = dot(z_ref[...], w1z[...])
    acc += dot(hdot(one_p, ptab[...]), w1p[...])
    acc += dot(hdot(one_t, ttab[...]), w1t[...])
    h1 = jnp.maximum(acc + b1r[...], 0.0)
    h2 = jnp.maximum(dot(h1, w2[...]) + b2r[...], 0.0)
    o_ref[...] = dot(h2, w3[...]) + b3r[...]


def _tc_mlp(z, ip, it, w1z, ptab, w1p, ttab, w1t, b1, w2, b2, w3, b3):
    grid = (_B // _TB,)
    batch_spec = lambda cols: pl.BlockSpec((_TB, cols), lambda i: (i, 0))
    full = lambda a: pl.BlockSpec(a.shape, lambda i: (0, 0))
    return pl.pallas_call(
        _mlp_body,
        grid=grid,
        in_specs=[batch_spec(_ZD), batch_spec(1), batch_spec(1),
                  full(w1z), full(ptab), full(w1p), full(ttab), full(w1t),
                  full(b1), full(w2), full(b2), full(w3), full(b3)],
        out_specs=batch_spec(64),
        out_shape=jax.ShapeDtypeStruct((_B, 64), jnp.float32),
    )(z, ip, it, w1z, ptab, w1p, ttab, w1t, b1, w2, b2, w3, b3)


def kernel(zip_code_id, project_type_id, trade_needed_id, zip_emb, pt_emb,
           tr_emb, W1, b1, W2, b2, W3, b3):
    ztab = jnp.pad(zip_emb, ((0, 0), (0, _ZD - 24)))
    iz = zip_code_id.astype(jnp.int32).reshape(_B // _CH, _CH)
    z = _sc_gather(ztab, iz)
    ip = project_type_id.astype(jnp.int32).reshape(_B, 1)
    it = trade_needed_id.astype(jnp.int32).reshape(_B, 1)
    w1z = jnp.pad(W1[0:24], ((0, _ZD - 24), (0, 0)))
    ptab = jnp.pad(pt_emb, ((0, 16 - 9), (0, 16 - 12)))
    w1p = jnp.pad(W1[24:36], ((0, 4), (0, 0)))
    ttab = jnp.pad(tr_emb, ((0, 16 - 11), (0, 0)))
    w1t = W1[36:52]
    return _tc_mlp(z, ip, it, w1z, ptab, w1p, ttab, w1t,
                   b1.reshape(1, 512), W2, b2.reshape(1, 128), W3,
                   b3.reshape(1, 64))


# speedup vs baseline: 3.3118x; 1.3116x over previous
import functools

import jax
import jax.numpy as jnp
from jax import lax
from jax.experimental import pallas as pl
from jax.experimental.pallas import tpu as pltpu
from jax.experimental.pallas import tpu_sc as plsc

_B = 16384
_ZD = 128
_NC, _NS = 2, 16
_NW = _NC * _NS
_BPW = _B // _NW
_CH = 128
_NCH = _BPW // _CH


def _sc_gather(ztab, iz):
    mesh = plsc.VectorSubcoreMesh(core_axis_name="c", subcore_axis_name="s")

    @functools.partial(
        pl.kernel, mesh=mesh,
        out_type=jax.ShapeDtypeStruct((_B, _ZD), jnp.float32),
        scratch_types=[
            pltpu.VMEM((_NCH, _CH), jnp.int32),
            pltpu.VMEM((_BPW, _ZD), jnp.float32),
            pltpu.SemaphoreType.DMA,
        ])
    def k(ztab_h, iz_h, zo_h, ziv, zrows, sem):
        wid = lax.axis_index("s") * _NC + lax.axis_index("c")
        base = wid * _BPW
        pltpu.sync_copy(iz_h.at[pl.ds(wid * _NCH, _NCH)], ziv)
        handles = [
            pltpu.async_copy(ztab_h.at[ziv.at[j]],
                             zrows.at[pl.ds(j * _CH, _CH)], sem)
            for j in range(_NCH)
        ]
        for h in handles:
            h.wait()
        pltpu.sync_copy(zrows, zo_h.at[pl.ds(base, _BPW)])

    return k(ztab, iz)


_TB = 4096


def _mlp_body(z_ref, ip_ref, it_ref, w1z, pttab, w1pt, b1r, w2, b2r, w3,
              b3r, o_ref):
    dot = functools.partial(jnp.dot, preferred_element_type=jnp.float32,
                            precision=lax.Precision.DEFAULT)
    iota32 = lax.broadcasted_iota(jnp.int32, (1, 32), 1)
    sel = jnp.where(iota32 < 16, ip_ref[...], it_ref[...] + 16)
    one_pt = (sel == iota32).astype(jnp.float32)
    acc = dot(z_ref[...], w1z[...])
    acc += dot(dot(one_pt, pttab[...]), w1pt[...])
    h1 = jnp.maximum(acc + b1r[...], 0.0)
    h2 = jnp.maximum(dot(h1, w2[...]) + b2r[...], 0.0)
    o_ref[...] = dot(h2, w3[...]) + b3r[...]


def _tc_mlp(z, ip, it, w1z, pttab, w1pt, b1, w2, b2, w3, b3):
    grid = (_B // _TB,)
    batch_spec = lambda cols: pl.BlockSpec((_TB, cols), lambda i: (i, 0))
    full = lambda a: pl.BlockSpec(a.shape, lambda i: (0, 0))
    return pl.pallas_call(
        _mlp_body,
        grid=grid,
        in_specs=[batch_spec(_ZD), batch_spec(1), batch_spec(1),
                  full(w1z), full(pttab), full(w1pt),
                  full(b1), full(w2), full(b2), full(w3), full(b3)],
        out_specs=batch_spec(64),
        out_shape=jax.ShapeDtypeStruct((_B, 64), jnp.float32),
    )(z, ip, it, w1z, pttab, w1pt, b1, w2, b2, w3, b3)


def kernel(zip_code_id, project_type_id, trade_needed_id, zip_emb, pt_emb,
           tr_emb, W1, b1, W2, b2, W3, b3):
    ztab = jnp.pad(zip_emb, ((0, 0), (0, _ZD - 24)))
    iz = zip_code_id.astype(jnp.int32).reshape(_B // _CH, _CH)
    z = _sc_gather(ztab, iz)
    ip = project_type_id.astype(jnp.int32).reshape(_B, 1)
    it = trade_needed_id.astype(jnp.int32).reshape(_B, 1)
    w1z = jnp.pad(W1[0:24], ((0, _ZD - 24), (0, 0)))
    pttab = jnp.zeros((32, 28), jnp.float32)
    pttab = pttab.at[0:9, 0:12].set(pt_emb).at[16:27, 12:28].set(tr_emb)
    w1pt = W1[24:52]
    return _tc_mlp(z, ip, it, w1z, pttab, w1pt,
                   b1.reshape(1, 512), W2, b2.reshape(1, 128), W3,
                   b3.reshape(1, 64))


# speedup vs baseline: 3.8087x; 1.1500x over previous
import functools

import jax
import jax.numpy as jnp
from jax import lax
from jax.experimental import pallas as pl
from jax.experimental.pallas import tpu as pltpu
from jax.experimental.pallas import tpu_sc as plsc

_B = 16384
_ZD = 128
_NC, _NS = 2, 16
_NW = _NC * _NS
_BPW = _B // _NW
_CH = 128
_NCH = _BPW // _CH


def _sc_gather(ztab, iz):
    mesh = plsc.VectorSubcoreMesh(core_axis_name="c", subcore_axis_name="s")

    @functools.partial(
        pl.kernel, mesh=mesh,
        out_type=jax.ShapeDtypeStruct((_B, _ZD), jnp.float32),
        scratch_types=[
            pltpu.VMEM((_NCH, _CH), jnp.int32),
            pltpu.VMEM((_BPW, _ZD), jnp.float32),
            [pltpu.SemaphoreType.DMA] * _NCH,
            pltpu.SemaphoreType.DMA,
        ])
    def k(ztab_h, iz_h, zo_h, ziv, zrows, gsems, wsem):
        wid = lax.axis_index("s") * _NC + lax.axis_index("c")
        base = wid * _BPW
        pltpu.sync_copy(iz_h.at[pl.ds(wid * _NCH, _NCH)], ziv)
        handles = [
            pltpu.async_copy(ztab_h.at[ziv.at[j]],
                             zrows.at[pl.ds(j * _CH, _CH)], gsems[j])
            for j in range(_NCH)
        ]
        for j in range(_NCH):
            handles[j].wait()
            pltpu.async_copy(zrows.at[pl.ds(j * _CH, _CH)],
                             zo_h.at[pl.ds(base + j * _CH, _CH)], wsem)
        for j in range(_NCH):
            pltpu.make_async_copy(
                zrows.at[pl.ds(j * _CH, _CH)],
                zo_h.at[pl.ds(base + j * _CH, _CH)], wsem).wait()

    return k(ztab, iz)


_TB = 4096


def _mlp_body(z_ref, ip_ref, it_ref, w1z, pttab, w1pt, b1r, w2, b2r, w3,
              b3r, o_ref):
    dot = functools.partial(jnp.dot, preferred_element_type=jnp.float32,
                            precision=lax.Precision.DEFAULT)
    iota32 = lax.broadcasted_iota(jnp.int32, (32, 1), 0)
    sel = jnp.where(iota32 < 16, ip_ref[0], it_ref[0] + 16)
    one_ptt = (sel == iota32).astype(jnp.float32)
    acc = dot(z_ref[...], w1z[...])
    acc += lax.dot_general(one_ptt, dot(pttab[...], w1pt[...]),
                           (((0,), (0,)), ((), ())),
                           precision=lax.Precision.DEFAULT,
                           preferred_element_type=jnp.float32)
    h1 = jnp.maximum(acc + b1r[...], 0.0)
    h2 = jnp.maximum(dot(h1, w2[...]) + b2r[...], 0.0)
    o_ref[...] = dot(h2, w3[...]) + b3r[...]


def _tc_mlp(z, ip, it, w1z, pttab, w1pt, b1, w2, b2, w3, b3):
    grid = (_B // _TB,)
    batch_spec = lambda cols: pl.BlockSpec((_TB, cols), lambda i: (i, 0))
    idx_spec = pl.BlockSpec((1, 1, _TB), lambda i: (i, 0, 0))
    full = lambda a: pl.BlockSpec(a.shape, lambda i: (0, 0))
    return pl.pallas_call(
        _mlp_body,
        grid=grid,
        in_specs=[batch_spec(_ZD), idx_spec, idx_spec,
                  full(w1z), full(pttab), full(w1pt),
                  full(b1), full(w2), full(b2), full(w3), full(b3)],
        out_specs=batch_spec(64),
        out_shape=jax.ShapeDtypeStruct((_B, 64), jnp.float32),
    )(z, ip, it, w1z, pttab, w1pt, b1, w2, b2, w3, b3)


def kernel(zip_code_id, project_type_id, trade_needed_id, zip_emb, pt_emb,
           tr_emb, W1, b1, W2, b2, W3, b3):
    ztab = jnp.pad(zip_emb, ((0, 0), (0, _ZD - 24)))
    iz = zip_code_id.astype(jnp.int32).reshape(_B // _CH, _CH)
    z = _sc_gather(ztab, iz)
    ip = project_type_id.astype(jnp.int32).reshape(_B // _TB, 1, _TB)
    it = trade_needed_id.astype(jnp.int32).reshape(_B // _TB, 1, _TB)
    w1z = jnp.pad(W1[0:24], ((0, _ZD - 24), (0, 0)))
    pttab = jnp.zeros((32, 28), jnp.float32)
    pttab = pttab.at[0:9, 0:12].set(pt_emb).at[16:27, 12:28].set(tr_emb)
    w1pt = W1[24:52]
    return _tc_mlp(z, ip, it, w1z, pttab, w1pt,
                   b1.reshape(1, 512), W2, b2.reshape(1, 128), W3,
                   b3.reshape(1, 64))
